# Initial kernel scaffold; baseline (speedup 1.0000x reference)
import jax
import jax.numpy as jnp
from jax.experimental import pallas as pl


def kernel(x, theta, bn_gamma, bn_beta):
    raise NotImplementedError("write your pallas kernel here")


# speedup vs baseline: 118.8547x; 118.8547x over previous
import jax
import jax.numpy as jnp
from jax.experimental import pallas as pl

L = 1024
C = 768
OUT = 159
EPS = 1e-5


def _hconv_kernel(x_ref, theta_ref, out_ref):
    xb = x_ref[...]
    xsq = jnp.sum(xb * xb, axis=1)
    g = jnp.dot(xb, xb.T, preferred_element_type=jnp.float32)
    dis = xsq[:, None] - 2.0 * g + xsq[None, :]

    col = jax.lax.broadcasted_iota(jnp.int32, (L, L), 1)
    m1 = jnp.min(dis, axis=1)
    i1 = jnp.min(jnp.where(dis == m1[:, None], col, L), axis=1)
    dis2 = jnp.where(col == i1[:, None], jnp.inf, dis)
    m2 = jnp.min(dis2, axis=1)
    i2 = jnp.min(jnp.where(dis2 == m2[:, None], col, L), axis=1)

    viota = jax.lax.broadcasted_iota(jnp.int32, (L, L), 0)
    eiota = col
    h = ((viota == i1[None, :]) | (viota == i2[None, :]) | (viota == eiota))
    h = h.astype(jnp.float32)

    coldeg = jnp.sum(h, axis=0)
    rowdeg = jnp.sum(h, axis=1)

    xt = jnp.dot(xb, theta_ref[...], preferred_element_type=jnp.float32)
    he = h * (1.0 / coldeg)[None, :]
    edge_ft = jnp.dot(he.T, xt, preferred_element_type=jnp.float32)
    hn = h * (1.0 / rowdeg)[:, None]
    out_ref[...] = jnp.dot(hn, edge_ft, preferred_element_type=jnp.float32)


def _bn_relu_kernel(y_ref, gamma_ref, beta_ref, out_ref):
    y = y_ref[...]
    n = y.shape[0] * y.shape[2]
    mean = jnp.sum(y, axis=(0, 2), keepdims=True) / n
    d = y - mean
    var = jnp.sum(d * d, axis=(0, 2), keepdims=True) / n
    yn = d * jax.lax.rsqrt(var + EPS)
    g = gamma_ref[...][None, :, :]
    b = beta_ref[...][None, :, :]
    out_ref[...] = jnp.maximum(yn * g + b, 0.0)


def kernel(x, theta, bn_gamma, bn_beta):
    b = x.shape[0]
    xr = x.reshape(b, L, C)

    out = pl.pallas_call(
        _hconv_kernel,
        grid=(b,),
        in_specs=[
            pl.BlockSpec((None, L, C), lambda i: (i, 0, 0)),
            pl.BlockSpec((C, OUT), lambda i: (0, 0)),
        ],
        out_specs=pl.BlockSpec((None, L, OUT), lambda i: (i, 0, 0)),
        out_shape=jax.ShapeDtypeStruct((b, L, OUT), jnp.float32),
    )(xr, theta)

    y = out.reshape(b, OUT, L)
    y = pl.pallas_call(
        _bn_relu_kernel,
        out_shape=jax.ShapeDtypeStruct((b, OUT, L), jnp.float32),
    )(y, bn_gamma.reshape(OUT, 1), bn_beta.reshape(OUT, 1))
    return y.reshape(b, OUT, L, 1)
